# baseline (speedup 1.0000x reference)
import jax
import jax.numpy as jnp
from jax import lax
from jax.experimental import pallas as pl
from jax.experimental.pallas import tpu as pltpu
from jax.experimental.pallas import tpu_sc as plsc

N = 10000
E = 320000
D_IN = 128
D_HID = 256
D_OUT = 128

C = 128
IB = 16
E_PAD = 327680
NCHUNK = E_PAD // C
CPW_ES = NCHUNK // 32
CPW_CS = NCHUNK // 16
ACC_ROWS = 10112
ZROWS = ACC_ROWS // 16
OROWS = ACC_ROWS // 16

_MESH = plsc.VectorSubcoreMesh(
    core_axis_name="c", subcore_axis_name="s", num_cores=2, num_subcores=16)


def _agg_pass(y_hbm, srcs, dsts, acc, src_v, dst_v, rows0, rows1,
              sem0, sem1, base_chunk, nblocks):
  def outer(g, carry):
    blk = pl.multiple_of(base_chunk + g * IB, IB)
    pltpu.sync_copy(srcs.at[pl.ds(blk, IB)], src_v)
    pltpu.sync_copy(dsts.at[pl.ds(blk, IB)], dst_v)
    pltpu.async_copy(y_hbm.at[src_v.at[0]], rows0, sem0)

    def inner(p, carry2):
      j0 = 2 * p
      pltpu.async_copy(y_hbm.at[src_v.at[j0 + 1]], rows1, sem1)
      pltpu.make_async_copy(y_hbm.at[src_v.at[j0]], rows0, sem0).wait()
      pltpu.sync_copy(rows0, acc.at[dst_v.at[j0]], add=True)

      @pl.when(p < IB // 2 - 1)
      def _():
        pltpu.async_copy(y_hbm.at[src_v.at[j0 + 2]], rows0, sem0)

      pltpu.make_async_copy(y_hbm.at[src_v.at[j0]], rows1, sem1).wait()
      pltpu.sync_copy(rows1, acc.at[dst_v.at[j0 + 1]], add=True)
      return carry2

    lax.fori_loop(0, IB // 2, inner, 0)
    return carry

  lax.fori_loop(0, nblocks, outer, 0)


def _zero_acc(zinit, acc, s):
  pltpu.sync_copy(zinit.at[pl.ds(s * ZROWS, ZROWS)],
                  acc.at[pl.ds(s * ZROWS, ZROWS)])


def _writeout(acc, o, s):
  pltpu.sync_copy(acc.at[pl.ds(s * OROWS, OROWS)],
                  o.at[pl.ds(s * OROWS, OROWS)])


def _agg_edge_split_body(x_hbm, srcs, dsts, zinit, q0, q1,
                         src_v, dst_v, rows0, rows1, acc, sem0, sem1):
  c = lax.axis_index("c")
  s = lax.axis_index("s")
  w = c * 16 + s
  _zero_acc(zinit, acc, s)
  plsc.subcore_barrier()
  _agg_pass(x_hbm, srcs, dsts, acc, src_v, dst_v, rows0, rows1, sem0, sem1,
            w * CPW_ES, CPW_ES // IB)
  plsc.subcore_barrier()

  @pl.when(c == 0)
  def _():
    _writeout(acc, q0, s)

  @pl.when(c == 1)
  def _():
    _writeout(acc, q1, s)


def _agg_col_split_body(ya, yb, srcs, dsts, zinit, a0, a1,
                        src_v, dst_v, rows0, rows1, acc, sem0, sem1):
  c = lax.axis_index("c")
  s = lax.axis_index("s")
  _zero_acc(zinit, acc, s)
  plsc.subcore_barrier()

  @pl.when(c == 0)
  def _():
    _agg_pass(ya, srcs, dsts, acc, src_v, dst_v, rows0, rows1, sem0, sem1,
              s * CPW_CS, CPW_CS // IB)

  @pl.when(c == 1)
  def _():
    _agg_pass(yb, srcs, dsts, acc, src_v, dst_v, rows0, rows1, sem0, sem1,
              s * CPW_CS, CPW_CS // IB)

  plsc.subcore_barrier()

  @pl.when(c == 0)
  def _():
    _writeout(acc, a0, s)

  @pl.when(c == 1)
  def _():
    _writeout(acc, a1, s)


def _make_agg(body):
  return pl.kernel(
      body,
      out_type=(jax.ShapeDtypeStruct((ACC_ROWS, 128), jnp.float32),
                jax.ShapeDtypeStruct((ACC_ROWS, 128), jnp.float32)),
      mesh=_MESH,
      scratch_types=(
          pltpu.VMEM((IB, C), jnp.int32),
          pltpu.VMEM((IB, C), jnp.int32),
          pltpu.VMEM((C, 128), jnp.float32),
          pltpu.VMEM((C, 128), jnp.float32),
          pltpu.VMEM_SHARED((ACC_ROWS, 128), jnp.float32),
          pltpu.SemaphoreType.DMA,
          pltpu.SemaphoreType.DMA,
      ),
  )


_agg_edge_split = _make_agg(_agg_edge_split_body)
_agg_col_split = _make_agg(_agg_col_split_body)

RB = 1000


def _mm1_body(p0, p1, w1, b1, ya, yb):
  h = p0[...] + p1[...]
  y = jnp.dot(h, w1[...], preferred_element_type=jnp.float32) + b1[...]
  y = jnp.maximum(y, 0.0)
  ya[...] = y[:, :128]
  yb[...] = y[:, 128:]


_mm1 = pl.pallas_call(
    _mm1_body,
    grid=(N // RB,),
    in_specs=[
        pl.BlockSpec((RB, D_IN), lambda i: (i, 0)),
        pl.BlockSpec((RB, D_IN), lambda i: (i, 0)),
        pl.BlockSpec((D_IN, D_HID), lambda i: (0, 0)),
        pl.BlockSpec((1, D_HID), lambda i: (0, 0)),
    ],
    out_specs=[
        pl.BlockSpec((RB, 128), lambda i: (i, 0)),
        pl.BlockSpec((RB, 128), lambda i: (i, 0)),
    ],
    out_shape=[
        jax.ShapeDtypeStruct((N, 128), jnp.float32),
        jax.ShapeDtypeStruct((N, 128), jnp.float32),
    ],
)


def _mm2_body(a0, a1, w2a, w2b, b2, w3, t3):
  h = (jnp.dot(a0[...], w2a[...], preferred_element_type=jnp.float32)
       + jnp.dot(a1[...], w2b[...], preferred_element_type=jnp.float32)
       + b2[...])
  h = jnp.maximum(h, 0.0)
  t3[...] = jnp.dot(h, w3[...], preferred_element_type=jnp.float32)


_mm2 = pl.pallas_call(
    _mm2_body,
    grid=(N // RB,),
    in_specs=[
        pl.BlockSpec((RB, 128), lambda i: (i, 0)),
        pl.BlockSpec((RB, 128), lambda i: (i, 0)),
        pl.BlockSpec((128, D_HID), lambda i: (0, 0)),
        pl.BlockSpec((128, D_HID), lambda i: (0, 0)),
        pl.BlockSpec((1, D_HID), lambda i: (0, 0)),
        pl.BlockSpec((D_HID, D_OUT), lambda i: (0, 0)),
    ],
    out_specs=pl.BlockSpec((RB, D_OUT), lambda i: (i, 0)),
    out_shape=jax.ShapeDtypeStruct((N, D_OUT), jnp.float32),
)


def _mm3_body(q0, q1, b3, out):
  out[...] = q0[...] + q1[...] + b3[...]


_mm3 = pl.pallas_call(
    _mm3_body,
    grid=(N // RB,),
    in_specs=[
        pl.BlockSpec((RB, D_OUT), lambda i: (i, 0)),
        pl.BlockSpec((RB, D_OUT), lambda i: (i, 0)),
        pl.BlockSpec((1, D_OUT), lambda i: (0, 0)),
    ],
    out_specs=pl.BlockSpec((RB, D_OUT), lambda i: (i, 0)),
    out_shape=jax.ShapeDtypeStruct((N, D_OUT), jnp.float32),
)


@jax.jit
def kernel(x, adj, W1, b1, W2, b2, W3, b3):
  src = adj[0].astype(jnp.int32)
  dst = adj[1].astype(jnp.int32)
  srcs = jnp.concatenate(
      [src, jnp.zeros((E_PAD - E,), jnp.int32)]).reshape(NCHUNK, C)
  pad_dst = N + jnp.arange(E_PAD - E, dtype=jnp.int32) % (ACC_ROWS - N)
  dsts = jnp.concatenate([dst, pad_dst]).reshape(NCHUNK, C)
  zinit = jnp.zeros((ACC_ROWS, 128), jnp.float32)

  q0, q1 = _agg_edge_split(x, srcs, dsts, zinit)
  ya, yb = _mm1(q0, q1, W1, b1.reshape(1, D_HID))
  a0, a1 = _agg_col_split(ya, yb, srcs, dsts, zinit)
  t3 = _mm2(a0, a1, W2[:128], W2[128:], b2.reshape(1, D_HID), W3)
  r0, r1 = _agg_edge_split(t3, srcs, dsts, zinit)
  return _mm3(r0, r1, b3.reshape(1, D_OUT))


# speedup vs baseline: 1.0007x; 1.0007x over previous
import jax
import jax.numpy as jnp
from jax import lax
from jax.experimental import pallas as pl
from jax.experimental.pallas import tpu as pltpu
from jax.experimental.pallas import tpu_sc as plsc

N = 10000
E = 320000
D_IN = 128
D_HID = 256
D_OUT = 128

C = 128
IB = 16
E_PAD = 327680
NCHUNK = E_PAD // C
CPW_ES = NCHUNK // 32
CPW_CS = NCHUNK // 16
ACC_ROWS = 10112
ZROWS = ACC_ROWS // 16
OROWS = ACC_ROWS // 16

_MESH = plsc.VectorSubcoreMesh(
    core_axis_name="c", subcore_axis_name="s", num_cores=2, num_subcores=16)


def _agg_pass(y_hbm, srcs, dsts, acc, src_v, dst_v, rows0, rows1,
              gsem0, gsem1, ssem0, ssem1, base_chunk, nblocks):
  def gwait(buf, gsem):
    pltpu.make_async_copy(y_hbm.at[src_v.at[0]], buf, gsem).wait()

  def swait(buf, ssem):
    pltpu.make_async_copy(buf, acc.at[dst_v.at[0]], ssem).wait()

  def outer(g, carry):
    blk = pl.multiple_of(base_chunk + g * IB, IB)
    pltpu.sync_copy(srcs.at[pl.ds(blk, IB)], src_v)
    pltpu.sync_copy(dsts.at[pl.ds(blk, IB)], dst_v)
    pltpu.async_copy(y_hbm.at[src_v.at[0]], rows0, gsem0)

    def inner(p, carry2):
      j0 = 2 * p

      @pl.when(p > 0)
      def _():
        swait(rows1, ssem1)

      pltpu.async_copy(y_hbm.at[src_v.at[j0 + 1]], rows1, gsem1)
      gwait(rows0, gsem0)
      pltpu.async_copy(rows0, acc.at[dst_v.at[j0]], ssem0, add=True)

      @pl.when(p < IB // 2 - 1)
      def _():
        swait(rows0, ssem0)
        pltpu.async_copy(y_hbm.at[src_v.at[j0 + 2]], rows0, gsem0)

      gwait(rows1, gsem1)
      pltpu.async_copy(rows1, acc.at[dst_v.at[j0 + 1]], ssem1, add=True)
      return carry2

    lax.fori_loop(0, IB // 2, inner, 0)
    swait(rows0, ssem0)
    swait(rows1, ssem1)
    return carry

  lax.fori_loop(0, nblocks, outer, 0)


def _zero_acc(zinit, acc, s):
  pltpu.sync_copy(zinit.at[pl.ds(s * ZROWS, ZROWS)],
                  acc.at[pl.ds(s * ZROWS, ZROWS)])


def _writeout(acc, o, s):
  pltpu.sync_copy(acc.at[pl.ds(s * OROWS, OROWS)],
                  o.at[pl.ds(s * OROWS, OROWS)])


def _agg_edge_split_body(x_hbm, srcs, dsts, zinit, q0, q1,
                         src_v, dst_v, rows0, rows1, acc,
                         gsem0, gsem1, ssem0, ssem1):
  c = lax.axis_index("c")
  s = lax.axis_index("s")
  w = c * 16 + s
  _zero_acc(zinit, acc, s)
  plsc.subcore_barrier()
  _agg_pass(x_hbm, srcs, dsts, acc, src_v, dst_v, rows0, rows1,
            gsem0, gsem1, ssem0, ssem1, w * CPW_ES, CPW_ES // IB)
  plsc.subcore_barrier()

  @pl.when(c == 0)
  def _():
    _writeout(acc, q0, s)

  @pl.when(c == 1)
  def _():
    _writeout(acc, q1, s)


def _agg_col_split_body(ya, yb, srcs, dsts, zinit, a0, a1,
                        src_v, dst_v, rows0, rows1, acc,
                        gsem0, gsem1, ssem0, ssem1):
  c = lax.axis_index("c")
  s = lax.axis_index("s")
  _zero_acc(zinit, acc, s)
  plsc.subcore_barrier()

  @pl.when(c == 0)
  def _():
    _agg_pass(ya, srcs, dsts, acc, src_v, dst_v, rows0, rows1,
              gsem0, gsem1, ssem0, ssem1, s * CPW_CS, CPW_CS // IB)

  @pl.when(c == 1)
  def _():
    _agg_pass(yb, srcs, dsts, acc, src_v, dst_v, rows0, rows1,
              gsem0, gsem1, ssem0, ssem1, s * CPW_CS, CPW_CS // IB)

  plsc.subcore_barrier()

  @pl.when(c == 0)
  def _():
    _writeout(acc, a0, s)

  @pl.when(c == 1)
  def _():
    _writeout(acc, a1, s)


def _make_agg(body):
  return pl.kernel(
      body,
      out_type=(jax.ShapeDtypeStruct((ACC_ROWS, 128), jnp.float32),
                jax.ShapeDtypeStruct((ACC_ROWS, 128), jnp.float32)),
      mesh=_MESH,
      scratch_types=(
          pltpu.VMEM((IB, C), jnp.int32),
          pltpu.VMEM((IB, C), jnp.int32),
          pltpu.VMEM((C, 128), jnp.float32),
          pltpu.VMEM((C, 128), jnp.float32),
          pltpu.VMEM_SHARED((ACC_ROWS, 128), jnp.float32),
          pltpu.SemaphoreType.DMA,
          pltpu.SemaphoreType.DMA,
          pltpu.SemaphoreType.DMA,
          pltpu.SemaphoreType.DMA,
      ),
  )


_agg_edge_split = _make_agg(_agg_edge_split_body)
_agg_col_split = _make_agg(_agg_col_split_body)

RB = 1000


def _mm1_body(p0, p1, w1, b1, ya, yb):
  h = p0[...] + p1[...]
  y = jnp.dot(h, w1[...], preferred_element_type=jnp.float32) + b1[...]
  y = jnp.maximum(y, 0.0)
  ya[...] = y[:, :128]
  yb[...] = y[:, 128:]


_mm1 = pl.pallas_call(
    _mm1_body,
    grid=(N // RB,),
    in_specs=[
        pl.BlockSpec((RB, D_IN), lambda i: (i, 0)),
        pl.BlockSpec((RB, D_IN), lambda i: (i, 0)),
        pl.BlockSpec((D_IN, D_HID), lambda i: (0, 0)),
        pl.BlockSpec((1, D_HID), lambda i: (0, 0)),
    ],
    out_specs=[
        pl.BlockSpec((RB, 128), lambda i: (i, 0)),
        pl.BlockSpec((RB, 128), lambda i: (i, 0)),
    ],
    out_shape=[
        jax.ShapeDtypeStruct((N, 128), jnp.float32),
        jax.ShapeDtypeStruct((N, 128), jnp.float32),
    ],
)


def _mm2_body(a0, a1, w2a, w2b, b2, w3, t3):
  h = (jnp.dot(a0[...], w2a[...], preferred_element_type=jnp.float32)
       + jnp.dot(a1[...], w2b[...], preferred_element_type=jnp.float32)
       + b2[...])
  h = jnp.maximum(h, 0.0)
  t3[...] = jnp.dot(h, w3[...], preferred_element_type=jnp.float32)


_mm2 = pl.pallas_call(
    _mm2_body,
    grid=(N // RB,),
    in_specs=[
        pl.BlockSpec((RB, 128), lambda i: (i, 0)),
        pl.BlockSpec((RB, 128), lambda i: (i, 0)),
        pl.BlockSpec((128, D_HID), lambda i: (0, 0)),
        pl.BlockSpec((128, D_HID), lambda i: (0, 0)),
        pl.BlockSpec((1, D_HID), lambda i: (0, 0)),
        pl.BlockSpec((D_HID, D_OUT), lambda i: (0, 0)),
    ],
    out_specs=pl.BlockSpec((RB, D_OUT), lambda i: (i, 0)),
    out_shape=jax.ShapeDtypeStruct((N, D_OUT), jnp.float32),
)


def _mm3_body(q0, q1, b3, out):
  out[...] = q0[...] + q1[...] + b3[...]


_mm3 = pl.pallas_call(
    _mm3_body,
    grid=(N // RB,),
    in_specs=[
        pl.BlockSpec((RB, D_OUT), lambda i: (i, 0)),
        pl.BlockSpec((RB, D_OUT), lambda i: (i, 0)),
        pl.BlockSpec((1, D_OUT), lambda i: (0, 0)),
    ],
    out_specs=pl.BlockSpec((RB, D_OUT), lambda i: (i, 0)),
    out_shape=jax.ShapeDtypeStruct((N, D_OUT), jnp.float32),
)


@jax.jit
def kernel(x, adj, W1, b1, W2, b2, W3, b3):
  src = adj[0].astype(jnp.int32)
  dst = adj[1].astype(jnp.int32)
  srcs = jnp.concatenate(
      [src, jnp.zeros((E_PAD - E,), jnp.int32)]).reshape(NCHUNK, C)
  pad_dst = N + jnp.arange(E_PAD - E, dtype=jnp.int32) % (ACC_ROWS - N)
  dsts = jnp.concatenate([dst, pad_dst]).reshape(NCHUNK, C)
  zinit = jnp.zeros((ACC_ROWS, 128), jnp.float32)

  q0, q1 = _agg_edge_split(x, srcs, dsts, zinit)
  ya, yb = _mm1(q0, q1, W1, b1.reshape(1, D_HID))
  a0, a1 = _agg_col_split(ya, yb, srcs, dsts, zinit)
  t3 = _mm2(a0, a1, W2[:128], W2[128:], b2.reshape(1, D_HID), W3)
  r0, r1 = _agg_edge_split(t3, srcs, dsts, zinit)
  return _mm3(r0, r1, b3.reshape(1, D_OUT))


# speedup vs baseline: 3.0612x; 3.0591x over previous
import jax
import jax.numpy as jnp
from jax import lax
from jax.experimental import pallas as pl
from jax.experimental.pallas import tpu as pltpu
from jax.experimental.pallas import tpu_sc as plsc

N = 10000
E = 320000
D_IN = 128
D_HID = 256
D_OUT = 128

C = 128
IB = 16
E_PAD = 327680
NCHUNK = E_PAD // C
CPW_ES = NCHUNK // 32
CPW_CS = NCHUNK // 16
ACC_ROWS = 10112
ZROWS = ACC_ROWS // 16
OROWS = ACC_ROWS // 16

_MESH = plsc.VectorSubcoreMesh(
    core_axis_name="c", subcore_axis_name="s", num_cores=2, num_subcores=16)


def _agg_pass(y_hbm, srcs, dsts, acc, src_v, dst_v, rows0, rows1,
              gsem0, gsem1, ssem0, ssem1, base_chunk, nblocks):
  def gwait(buf, gsem):
    pltpu.make_async_copy(y_hbm.at[src_v.at[0]], buf, gsem).wait()

  def swait(buf, ssem):
    pltpu.make_async_copy(buf, acc.at[dst_v.at[0]], ssem).wait()

  def outer(g, carry):
    blk = pl.multiple_of(base_chunk + g * IB, IB)
    pltpu.sync_copy(srcs.at[pl.ds(blk, IB)], src_v)
    pltpu.sync_copy(dsts.at[pl.ds(blk, IB)], dst_v)
    pltpu.async_copy(y_hbm.at[src_v.at[0]], rows0, gsem0)

    def inner(p, carry2):
      j0 = 2 * p

      @pl.when(p > 0)
      def _():
        swait(rows1, ssem1)

      pltpu.async_copy(y_hbm.at[src_v.at[j0 + 1]], rows1, gsem1)
      gwait(rows0, gsem0)
      pltpu.async_copy(rows0, acc.at[dst_v.at[j0]], ssem0, add=True)

      @pl.when(p < IB // 2 - 1)
      def _():
        swait(rows0, ssem0)
        pltpu.async_copy(y_hbm.at[src_v.at[j0 + 2]], rows0, gsem0)

      gwait(rows1, gsem1)
      pltpu.async_copy(rows1, acc.at[dst_v.at[j0 + 1]], ssem1, add=True)
      return carry2

    lax.fori_loop(0, IB // 2, inner, 0)
    swait(rows0, ssem0)
    swait(rows1, ssem1)
    return carry

  lax.fori_loop(0, nblocks, outer, 0)


def _zero_acc(zinit, acc, s):
  pltpu.sync_copy(zinit.at[pl.ds(s * ZROWS, ZROWS)],
                  acc.at[pl.ds(s * ZROWS, ZROWS)])


def _writeout(acc, o, s):
  pltpu.sync_copy(acc.at[pl.ds(s * OROWS, OROWS)],
                  o.at[pl.ds(s * OROWS, OROWS)])


def _agg_edge_split_body(x_hbm, srcs, dsts, zinit, q0, q1,
                         src_v, dst_v, rows0, rows1, acc,
                         gsem0, gsem1, ssem0, ssem1):
  c = lax.axis_index("c")
  s = lax.axis_index("s")
  w = c * 16 + s
  _zero_acc(zinit, acc, s)
  plsc.subcore_barrier()
  _agg_pass(x_hbm, srcs, dsts, acc, src_v, dst_v, rows0, rows1,
            gsem0, gsem1, ssem0, ssem1, w * CPW_ES, CPW_ES // IB)
  plsc.subcore_barrier()

  @pl.when(c == 0)
  def _():
    _writeout(acc, q0, s)

  @pl.when(c == 1)
  def _():
    _writeout(acc, q1, s)


def _agg_col_split_body(ya, yb, srcs, dsts, zinit, a0, a1,
                        src_v, dst_v, rows0, rows1, acc,
                        gsem0, gsem1, ssem0, ssem1):
  c = lax.axis_index("c")
  s = lax.axis_index("s")
  _zero_acc(zinit, acc, s)
  plsc.subcore_barrier()

  @pl.when(c == 0)
  def _():
    _agg_pass(ya, srcs, dsts, acc, src_v, dst_v, rows0, rows1,
              gsem0, gsem1, ssem0, ssem1, s * CPW_CS, CPW_CS // IB)

  @pl.when(c == 1)
  def _():
    _agg_pass(yb, srcs, dsts, acc, src_v, dst_v, rows0, rows1,
              gsem0, gsem1, ssem0, ssem1, s * CPW_CS, CPW_CS // IB)

  plsc.subcore_barrier()

  @pl.when(c == 0)
  def _():
    _writeout(acc, a0, s)

  @pl.when(c == 1)
  def _():
    _writeout(acc, a1, s)


def _make_agg(body):
  return pl.kernel(
      body,
      out_type=(jax.ShapeDtypeStruct((ACC_ROWS, 128), jnp.float32),
                jax.ShapeDtypeStruct((ACC_ROWS, 128), jnp.float32)),
      mesh=_MESH,
      scratch_types=(
          pltpu.VMEM((IB, C), jnp.int32),
          pltpu.VMEM((IB, C), jnp.int32),
          pltpu.VMEM((C, 128), jnp.float32),
          pltpu.VMEM((C, 128), jnp.float32),
          pltpu.VMEM_SHARED((ACC_ROWS, 128), jnp.float32),
          pltpu.SemaphoreType.DMA,
          pltpu.SemaphoreType.DMA,
          pltpu.SemaphoreType.DMA,
          pltpu.SemaphoreType.DMA,
      ),
  )


_agg_edge_split = _make_agg(_agg_edge_split_body)
_agg_col_split = _make_agg(_agg_col_split_body)

RB = 1000


def _mm1_body(p0, p1, w1, b1, ya, yb):
  h = p0[...] + p1[...]
  y = jnp.dot(h, w1[...], preferred_element_type=jnp.float32) + b1[...]
  y = jnp.maximum(y, 0.0)
  ya[...] = y[:, :128]
  yb[...] = y[:, 128:]


_mm1 = pl.pallas_call(
    _mm1_body,
    grid=(N // RB,),
    in_specs=[
        pl.BlockSpec((RB, D_IN), lambda i: (i, 0)),
        pl.BlockSpec((RB, D_IN), lambda i: (i, 0)),
        pl.BlockSpec((D_IN, D_HID), lambda i: (0, 0)),
        pl.BlockSpec((1, D_HID), lambda i: (0, 0)),
    ],
    out_specs=[
        pl.BlockSpec((RB, 128), lambda i: (i, 0)),
        pl.BlockSpec((RB, 128), lambda i: (i, 0)),
    ],
    out_shape=[
        jax.ShapeDtypeStruct((N, 128), jnp.float32),
        jax.ShapeDtypeStruct((N, 128), jnp.float32),
    ],
)


def _mm2_body(a0, a1, w2a, w2b, b2, w3, t3):
  h = (jnp.dot(a0[...], w2a[...], preferred_element_type=jnp.float32)
       + jnp.dot(a1[...], w2b[...], preferred_element_type=jnp.float32)
       + b2[...])
  h = jnp.maximum(h, 0.0)
  t3[...] = jnp.dot(h, w3[...], preferred_element_type=jnp.float32)


_mm2 = pl.pallas_call(
    _mm2_body,
    grid=(N // RB,),
    in_specs=[
        pl.BlockSpec((RB, 128), lambda i: (i, 0)),
        pl.BlockSpec((RB, 128), lambda i: (i, 0)),
        pl.BlockSpec((128, D_HID), lambda i: (0, 0)),
        pl.BlockSpec((128, D_HID), lambda i: (0, 0)),
        pl.BlockSpec((1, D_HID), lambda i: (0, 0)),
        pl.BlockSpec((D_HID, D_OUT), lambda i: (0, 0)),
    ],
    out_specs=pl.BlockSpec((RB, D_OUT), lambda i: (i, 0)),
    out_shape=jax.ShapeDtypeStruct((N, D_OUT), jnp.float32),
)


def _mm3_body(q0, q1, b3, out):
  out[...] = q0[...] + q1[...] + b3[...]


_mm3 = pl.pallas_call(
    _mm3_body,
    grid=(N // RB,),
    in_specs=[
        pl.BlockSpec((RB, D_OUT), lambda i: (i, 0)),
        pl.BlockSpec((RB, D_OUT), lambda i: (i, 0)),
        pl.BlockSpec((1, D_OUT), lambda i: (0, 0)),
    ],
    out_specs=pl.BlockSpec((RB, D_OUT), lambda i: (i, 0)),
    out_shape=jax.ShapeDtypeStruct((N, D_OUT), jnp.float32),
)


@jax.jit
def kernel(x, adj, W1, b1, W2, b2, W3, b3):
  src = adj[0].astype(jnp.int32)
  dst = adj[1].astype(jnp.int32)
  pad_src = jnp.arange(E_PAD - E, dtype=jnp.int32) % N
  srcs = jnp.concatenate([src, pad_src]).reshape(NCHUNK, C)
  pad_dst = N + jnp.arange(E_PAD - E, dtype=jnp.int32) % (ACC_ROWS - N)
  dsts = jnp.concatenate([dst, pad_dst]).reshape(NCHUNK, C)
  zinit = jnp.zeros((ACC_ROWS, 128), jnp.float32)

  q0, q1 = _agg_edge_split(x, srcs, dsts, zinit)
  ya, yb = _mm1(q0, q1, W1, b1.reshape(1, D_HID))
  a0, a1 = _agg_col_split(ya, yb, srcs, dsts, zinit)
  t3 = _mm2(a0, a1, W2[:128], W2[128:], b2.reshape(1, D_HID), W3)
  r0, r1 = _agg_edge_split(t3, srcs, dsts, zinit)
  return _mm3(r0, r1, b3.reshape(1, D_OUT))


# speedup vs baseline: 3.2617x; 1.0655x over previous
import jax
import jax.numpy as jnp
from jax import lax
from jax.experimental import pallas as pl
from jax.experimental.pallas import tpu as pltpu
from jax.experimental.pallas import tpu_sc as plsc

N = 10000
E = 320000
D_IN = 128
D_HID = 256
D_OUT = 128

C = 128
IB = 40
E_PAD = 327680
NCHUNK = E_PAD // C
CPW_ES = NCHUNK // 32
CPW_CS = NCHUNK // 16
ACC_ROWS = 10112
ZROWS = ACC_ROWS // 16
OROWS = ACC_ROWS // 16

_MESH = plsc.VectorSubcoreMesh(
    core_axis_name="c", subcore_axis_name="s", num_cores=2, num_subcores=16)


def _agg_pass(y_hbm, srcs, dsts, acc, src_v, dst_v, rows0, rows1,
              gsem0, gsem1, ssem0, ssem1, base_chunk, nblocks):
  def gwait(buf, gsem):
    pltpu.make_async_copy(y_hbm.at[src_v.at[0]], buf, gsem).wait()

  def swait(buf, ssem):
    pltpu.make_async_copy(buf, acc.at[dst_v.at[0]], ssem).wait()

  def outer(g, carry):
    blk = pl.multiple_of(base_chunk + g * IB, IB)
    pltpu.sync_copy(srcs.at[pl.ds(blk, IB)], src_v)
    pltpu.sync_copy(dsts.at[pl.ds(blk, IB)], dst_v)
    pltpu.async_copy(y_hbm.at[src_v.at[0]], rows0, gsem0)

    def inner(p, carry2):
      j0 = 2 * p

      @pl.when(p > 0)
      def _():
        swait(rows1, ssem1)

      pltpu.async_copy(y_hbm.at[src_v.at[j0 + 1]], rows1, gsem1)
      gwait(rows0, gsem0)
      pltpu.async_copy(rows0, acc.at[dst_v.at[j0]], ssem0, add=True)

      @pl.when(p < IB // 2 - 1)
      def _():
        swait(rows0, ssem0)
        pltpu.async_copy(y_hbm.at[src_v.at[j0 + 2]], rows0, gsem0)

      gwait(rows1, gsem1)
      pltpu.async_copy(rows1, acc.at[dst_v.at[j0 + 1]], ssem1, add=True)
      return carry2

    lax.fori_loop(0, IB // 2, inner, 0)
    swait(rows0, ssem0)
    swait(rows1, ssem1)
    return carry

  lax.fori_loop(0, nblocks, outer, 0)


def _zero_acc(zinit, acc, s):
  pltpu.sync_copy(zinit.at[pl.ds(s * ZROWS, ZROWS)],
                  acc.at[pl.ds(s * ZROWS, ZROWS)])


def _writeout(acc, o, s):
  pltpu.sync_copy(acc.at[pl.ds(s * OROWS, OROWS)],
                  o.at[pl.ds(s * OROWS, OROWS)])


def _agg_edge_split_body(x_hbm, srcs, dsts, zinit, q0, q1,
                         src_v, dst_v, rows0, rows1, acc,
                         gsem0, gsem1, ssem0, ssem1):
  c = lax.axis_index("c")
  s = lax.axis_index("s")
  w = c * 16 + s
  _zero_acc(zinit, acc, s)
  plsc.subcore_barrier()
  _agg_pass(x_hbm, srcs, dsts, acc, src_v, dst_v, rows0, rows1,
            gsem0, gsem1, ssem0, ssem1, w * CPW_ES, CPW_ES // IB)
  plsc.subcore_barrier()

  @pl.when(c == 0)
  def _():
    _writeout(acc, q0, s)

  @pl.when(c == 1)
  def _():
    _writeout(acc, q1, s)


def _agg_col_split_body(ya, yb, srcs, dsts, zinit, a0, a1,
                        src_v, dst_v, rows0, rows1, acc,
                        gsem0, gsem1, ssem0, ssem1):
  c = lax.axis_index("c")
  s = lax.axis_index("s")
  _zero_acc(zinit, acc, s)
  plsc.subcore_barrier()

  @pl.when(c == 0)
  def _():
    _agg_pass(ya, srcs, dsts, acc, src_v, dst_v, rows0, rows1,
              gsem0, gsem1, ssem0, ssem1, s * CPW_CS, CPW_CS // IB)

  @pl.when(c == 1)
  def _():
    _agg_pass(yb, srcs, dsts, acc, src_v, dst_v, rows0, rows1,
              gsem0, gsem1, ssem0, ssem1, s * CPW_CS, CPW_CS // IB)

  plsc.subcore_barrier()

  @pl.when(c == 0)
  def _():
    _writeout(acc, a0, s)

  @pl.when(c == 1)
  def _():
    _writeout(acc, a1, s)


def _make_agg(body):
  return pl.kernel(
      body,
      out_type=(jax.ShapeDtypeStruct((ACC_ROWS, 128), jnp.float32),
                jax.ShapeDtypeStruct((ACC_ROWS, 128), jnp.float32)),
      mesh=_MESH,
      scratch_types=(
          pltpu.VMEM((IB, C), jnp.int32),
          pltpu.VMEM((IB, C), jnp.int32),
          pltpu.VMEM((C, 128), jnp.float32),
          pltpu.VMEM((C, 128), jnp.float32),
          pltpu.VMEM_SHARED((ACC_ROWS, 128), jnp.float32),
          pltpu.SemaphoreType.DMA,
          pltpu.SemaphoreType.DMA,
          pltpu.SemaphoreType.DMA,
          pltpu.SemaphoreType.DMA,
      ),
  )


_agg_edge_split = _make_agg(_agg_edge_split_body)
_agg_col_split = _make_agg(_agg_col_split_body)

RB = 1000


def _mm1_body(p0, p1, w1, b1, ya, yb):
  h = p0[...] + p1[...]
  y = jnp.dot(h, w1[...], preferred_element_type=jnp.float32) + b1[...]
  y = jnp.maximum(y, 0.0)
  ya[...] = y[:, :128]
  yb[...] = y[:, 128:]


_mm1 = pl.pallas_call(
    _mm1_body,
    grid=(N // RB,),
    in_specs=[
        pl.BlockSpec((RB, D_IN), lambda i: (i, 0)),
        pl.BlockSpec((RB, D_IN), lambda i: (i, 0)),
        pl.BlockSpec((D_IN, D_HID), lambda i: (0, 0)),
        pl.BlockSpec((1, D_HID), lambda i: (0, 0)),
    ],
    out_specs=[
        pl.BlockSpec((RB, 128), lambda i: (i, 0)),
        pl.BlockSpec((RB, 128), lambda i: (i, 0)),
    ],
    out_shape=[
        jax.ShapeDtypeStruct((N, 128), jnp.float32),
        jax.ShapeDtypeStruct((N, 128), jnp.float32),
    ],
)


def _mm2_body(a0, a1, w2a, w2b, b2, w3, t3):
  h = (jnp.dot(a0[...], w2a[...], preferred_element_type=jnp.float32)
       + jnp.dot(a1[...], w2b[...], preferred_element_type=jnp.float32)
       + b2[...])
  h = jnp.maximum(h, 0.0)
  t3[...] = jnp.dot(h, w3[...], preferred_element_type=jnp.float32)


_mm2 = pl.pallas_call(
    _mm2_body,
    grid=(N // RB,),
    in_specs=[
        pl.BlockSpec((RB, 128), lambda i: (i, 0)),
        pl.BlockSpec((RB, 128), lambda i: (i, 0)),
        pl.BlockSpec((128, D_HID), lambda i: (0, 0)),
        pl.BlockSpec((128, D_HID), lambda i: (0, 0)),
        pl.BlockSpec((1, D_HID), lambda i: (0, 0)),
        pl.BlockSpec((D_HID, D_OUT), lambda i: (0, 0)),
    ],
    out_specs=pl.BlockSpec((RB, D_OUT), lambda i: (i, 0)),
    out_shape=jax.ShapeDtypeStruct((N, D_OUT), jnp.float32),
)


def _mm3_body(q0, q1, b3, out):
  out[...] = q0[...] + q1[...] + b3[...]


_mm3 = pl.pallas_call(
    _mm3_body,
    grid=(N // RB,),
    in_specs=[
        pl.BlockSpec((RB, D_OUT), lambda i: (i, 0)),
        pl.BlockSpec((RB, D_OUT), lambda i: (i, 0)),
        pl.BlockSpec((1, D_OUT), lambda i: (0, 0)),
    ],
    out_specs=pl.BlockSpec((RB, D_OUT), lambda i: (i, 0)),
    out_shape=jax.ShapeDtypeStruct((N, D_OUT), jnp.float32),
)


@jax.jit
def kernel(x, adj, W1, b1, W2, b2, W3, b3):
  src = adj[0].astype(jnp.int32)
  dst = adj[1].astype(jnp.int32)
  pad_src = jnp.arange(E_PAD - E, dtype=jnp.int32) % N
  srcs = jnp.concatenate([src, pad_src]).reshape(NCHUNK, C)
  pad_dst = N + jnp.arange(E_PAD - E, dtype=jnp.int32) % (ACC_ROWS - N)
  dsts = jnp.concatenate([dst, pad_dst]).reshape(NCHUNK, C)
  zinit = jnp.zeros((ACC_ROWS, 128), jnp.float32)

  q0, q1 = _agg_edge_split(x, srcs, dsts, zinit)
  ya, yb = _mm1(q0, q1, W1, b1.reshape(1, D_HID))
  a0, a1 = _agg_col_split(ya, yb, srcs, dsts, zinit)
  t3 = _mm2(a0, a1, W2[:128], W2[128:], b2.reshape(1, D_HID), W3)
  r0, r1 = _agg_edge_split(t3, srcs, dsts, zinit)
  return _mm3(r0, r1, b3.reshape(1, D_OUT))


# speedup vs baseline: 3.3784x; 1.0358x over previous
import jax
import jax.numpy as jnp
from jax import lax
from jax.experimental import pallas as pl
from jax.experimental.pallas import tpu as pltpu
from jax.experimental.pallas import tpu_sc as plsc

N = 10000
E = 320000
D_IN = 128
D_HID = 256
D_OUT = 128

C = 128
IB = 40
E_PAD = 327680
NCHUNK = E_PAD // C
CPW_ES = NCHUNK // 32
CPW_CS = NCHUNK // 16
ACC_ROWS = 10112
ZROWS = ACC_ROWS // 16
OROWS = ACC_ROWS // 16

_MESH = plsc.VectorSubcoreMesh(
    core_axis_name="c", subcore_axis_name="s", num_cores=2, num_subcores=16)


def _agg_pass(y_hbm, srcs, dsts, acc, src_v, dst_v, rows0, rows1,
              gsem0, gsem1, ssem0, ssem1, base_chunk, nblocks):
  def gwait(buf, gsem):
    pltpu.make_async_copy(y_hbm.at[src_v.at[0]], buf, gsem).wait()

  def swait(buf, ssem):
    pltpu.make_async_copy(buf, acc.at[dst_v.at[0]], ssem).wait()

  def outer(g, carry):
    blk = pl.multiple_of(base_chunk + g * IB, IB)
    pltpu.sync_copy(srcs.at[pl.ds(blk, IB)], src_v)
    pltpu.sync_copy(dsts.at[pl.ds(blk, IB)], dst_v)
    pltpu.async_copy(y_hbm.at[src_v.at[0]], rows0, gsem0)

    def inner(p, carry2):
      j0 = 2 * p

      @pl.when(p > 0)
      def _():
        swait(rows1, ssem1)

      pltpu.async_copy(y_hbm.at[src_v.at[j0 + 1]], rows1, gsem1)
      gwait(rows0, gsem0)
      pltpu.async_copy(rows0, acc.at[dst_v.at[j0]], ssem0, add=True)

      @pl.when(p < IB // 2 - 1)
      def _():
        swait(rows0, ssem0)
        pltpu.async_copy(y_hbm.at[src_v.at[j0 + 2]], rows0, gsem0)

      gwait(rows1, gsem1)
      pltpu.async_copy(rows1, acc.at[dst_v.at[j0 + 1]], ssem1, add=True)
      return carry2

    lax.fori_loop(0, IB // 2, inner, 0)
    swait(rows0, ssem0)
    swait(rows1, ssem1)
    return carry

  lax.fori_loop(0, nblocks, outer, 0)


def _zero_acc(rows0, acc, s):
  def zrow(r, carry):
    for k in range(8):
      rows0[r, pl.ds(k * 16, 16)] = jnp.zeros((16,), jnp.float32)
    return carry

  lax.fori_loop(0, C, zrow, 0)
  for k in range(4):
    pltpu.sync_copy(rows0, acc.at[pl.ds(s * ZROWS + k * C, C)])
  pltpu.sync_copy(rows0.at[pl.ds(0, ZROWS - 4 * C)],
                  acc.at[pl.ds(s * ZROWS + 4 * C, ZROWS - 4 * C)])


def _writeout(acc, o, s):
  pltpu.sync_copy(acc.at[pl.ds(s * OROWS, OROWS)],
                  o.at[pl.ds(s * OROWS, OROWS)])


def _agg_edge_split_body(x_hbm, srcs, dsts, q0, q1,
                         src_v, dst_v, rows0, rows1, acc,
                         gsem0, gsem1, ssem0, ssem1):
  c = lax.axis_index("c")
  s = lax.axis_index("s")
  w = c * 16 + s
  _zero_acc(rows0, acc, s)
  plsc.subcore_barrier()
  _agg_pass(x_hbm, srcs, dsts, acc, src_v, dst_v, rows0, rows1,
            gsem0, gsem1, ssem0, ssem1, w * CPW_ES, CPW_ES // IB)
  plsc.subcore_barrier()

  @pl.when(c == 0)
  def _():
    _writeout(acc, q0, s)

  @pl.when(c == 1)
  def _():
    _writeout(acc, q1, s)


def _agg_col_split_body(ya, yb, srcs, dsts, a0, a1,
                        src_v, dst_v, rows0, rows1, acc,
                        gsem0, gsem1, ssem0, ssem1):
  c = lax.axis_index("c")
  s = lax.axis_index("s")
  _zero_acc(rows0, acc, s)
  plsc.subcore_barrier()

  @pl.when(c == 0)
  def _():
    _agg_pass(ya, srcs, dsts, acc, src_v, dst_v, rows0, rows1,
              gsem0, gsem1, ssem0, ssem1, s * CPW_CS, CPW_CS // IB)

  @pl.when(c == 1)
  def _():
    _agg_pass(yb, srcs, dsts, acc, src_v, dst_v, rows0, rows1,
              gsem0, gsem1, ssem0, ssem1, s * CPW_CS, CPW_CS // IB)

  plsc.subcore_barrier()

  @pl.when(c == 0)
  def _():
    _writeout(acc, a0, s)

  @pl.when(c == 1)
  def _():
    _writeout(acc, a1, s)


def _make_agg(body):
  return pl.kernel(
      body,
      out_type=(jax.ShapeDtypeStruct((ACC_ROWS, 128), jnp.float32),
                jax.ShapeDtypeStruct((ACC_ROWS, 128), jnp.float32)),
      mesh=_MESH,
      scratch_types=(
          pltpu.VMEM((IB, C), jnp.int32),
          pltpu.VMEM((IB, C), jnp.int32),
          pltpu.VMEM((C, 128), jnp.float32),
          pltpu.VMEM((C, 128), jnp.float32),
          pltpu.VMEM_SHARED((ACC_ROWS, 128), jnp.float32),
          pltpu.SemaphoreType.DMA,
          pltpu.SemaphoreType.DMA,
          pltpu.SemaphoreType.DMA,
          pltpu.SemaphoreType.DMA,
      ),
  )


_agg_edge_split = _make_agg(_agg_edge_split_body)
_agg_col_split = _make_agg(_agg_col_split_body)

RB = 2000


def _mm1_body(p0, p1, w1, b1, ya, yb):
  h = p0[...] + p1[...]
  y = jnp.dot(h, w1[...], preferred_element_type=jnp.float32) + b1[...]
  y = jnp.maximum(y, 0.0)
  ya[...] = y[:, :128]
  yb[...] = y[:, 128:]


_mm1 = pl.pallas_call(
    _mm1_body,
    grid=(N // RB,),
    in_specs=[
        pl.BlockSpec((RB, D_IN), lambda i: (i, 0)),
        pl.BlockSpec((RB, D_IN), lambda i: (i, 0)),
        pl.BlockSpec((D_IN, D_HID), lambda i: (0, 0)),
        pl.BlockSpec((1, D_HID), lambda i: (0, 0)),
    ],
    out_specs=[
        pl.BlockSpec((RB, 128), lambda i: (i, 0)),
        pl.BlockSpec((RB, 128), lambda i: (i, 0)),
    ],
    out_shape=[
        jax.ShapeDtypeStruct((N, 128), jnp.float32),
        jax.ShapeDtypeStruct((N, 128), jnp.float32),
    ],
)


def _mm2_body(a0, a1, w2a, w2b, b2, w3, t3):
  h = (jnp.dot(a0[...], w2a[...], preferred_element_type=jnp.float32)
       + jnp.dot(a1[...], w2b[...], preferred_element_type=jnp.float32)
       + b2[...])
  h = jnp.maximum(h, 0.0)
  t3[...] = jnp.dot(h, w3[...], preferred_element_type=jnp.float32)


_mm2 = pl.pallas_call(
    _mm2_body,
    grid=(N // RB,),
    in_specs=[
        pl.BlockSpec((RB, 128), lambda i: (i, 0)),
        pl.BlockSpec((RB, 128), lambda i: (i, 0)),
        pl.BlockSpec((128, D_HID), lambda i: (0, 0)),
        pl.BlockSpec((128, D_HID), lambda i: (0, 0)),
        pl.BlockSpec((1, D_HID), lambda i: (0, 0)),
        pl.BlockSpec((D_HID, D_OUT), lambda i: (0, 0)),
    ],
    out_specs=pl.BlockSpec((RB, D_OUT), lambda i: (i, 0)),
    out_shape=jax.ShapeDtypeStruct((N, D_OUT), jnp.float32),
)


def _mm3_body(q0, q1, b3, out):
  out[...] = q0[...] + q1[...] + b3[...]


_mm3 = pl.pallas_call(
    _mm3_body,
    grid=(N // RB,),
    in_specs=[
        pl.BlockSpec((RB, D_OUT), lambda i: (i, 0)),
        pl.BlockSpec((RB, D_OUT), lambda i: (i, 0)),
        pl.BlockSpec((1, D_OUT), lambda i: (0, 0)),
    ],
    out_specs=pl.BlockSpec((RB, D_OUT), lambda i: (i, 0)),
    out_shape=jax.ShapeDtypeStruct((N, D_OUT), jnp.float32),
)


@jax.jit
def kernel(x, adj, W1, b1, W2, b2, W3, b3):
  src = adj[0].astype(jnp.int32)
  dst = adj[1].astype(jnp.int32)
  pad_src = jnp.arange(E_PAD - E, dtype=jnp.int32) % N
  srcs = jnp.concatenate([src, pad_src]).reshape(NCHUNK, C)
  pad_dst = N + jnp.arange(E_PAD - E, dtype=jnp.int32) % (ACC_ROWS - N)
  dsts = jnp.concatenate([dst, pad_dst]).reshape(NCHUNK, C)

  q0, q1 = _agg_edge_split(x, srcs, dsts)
  ya, yb = _mm1(q0, q1, W1, b1.reshape(1, D_HID))
  a0, a1 = _agg_col_split(ya, yb, srcs, dsts)
  t3 = _mm2(a0, a1, W2[:128], W2[128:], b2.reshape(1, D_HID), W3)
  r0, r1 = _agg_edge_split(t3, srcs, dsts)
  return _mm3(r0, r1, b3.reshape(1, D_OUT))


# speedup vs baseline: 3.3930x; 1.0043x over previous
import jax
import jax.numpy as jnp
from jax import lax
from jax.experimental import pallas as pl
from jax.experimental.pallas import tpu as pltpu
from jax.experimental.pallas import tpu_sc as plsc

N = 10000
E = 320000
D_IN = 128
D_HID = 256
D_OUT = 128

C = 128
IB = 40
NCHUNK = E // C
CPW_ES = 80
CPW_CS = 160
MAIN = 31 * CPW_ES
TAIL = 24
ACC_ROWS = 10112
ZROWS = ACC_ROWS // 16
OROWS = ACC_ROWS // 16

_MESH = plsc.VectorSubcoreMesh(
    core_axis_name="c", subcore_axis_name="s", num_cores=2, num_subcores=16)


def _agg_block(y_hbm, srcs, dsts, acc, src_v, dst_v, rows0, rows1,
               gsem0, gsem1, ssem0, ssem1, blk, ib):
  def gwait(buf, gsem):
    pltpu.make_async_copy(y_hbm.at[src_v.at[0]], buf, gsem).wait()

  def swait(buf, ssem):
    pltpu.make_async_copy(buf, acc.at[dst_v.at[0]], ssem).wait()

  blk = pl.multiple_of(blk, 8)
  pltpu.sync_copy(srcs.at[pl.ds(blk, ib)], src_v.at[pl.ds(0, ib)])
  pltpu.sync_copy(dsts.at[pl.ds(blk, ib)], dst_v.at[pl.ds(0, ib)])
  pltpu.async_copy(y_hbm.at[src_v.at[0]], rows0, gsem0)

  def inner(p, carry2):
    j0 = 2 * p

    @pl.when(p > 0)
    def _():
      swait(rows1, ssem1)

    pltpu.async_copy(y_hbm.at[src_v.at[j0 + 1]], rows1, gsem1)
    gwait(rows0, gsem0)
    pltpu.async_copy(rows0, acc.at[dst_v.at[j0]], ssem0, add=True)

    @pl.when(p < ib // 2 - 1)
    def _():
      swait(rows0, ssem0)
      pltpu.async_copy(y_hbm.at[src_v.at[j0 + 2]], rows0, gsem0)

    gwait(rows1, gsem1)
    pltpu.async_copy(rows1, acc.at[dst_v.at[j0 + 1]], ssem1, add=True)
    return carry2

  lax.fori_loop(0, ib // 2, inner, 0)
  swait(rows0, ssem0)
  swait(rows1, ssem1)


def _agg_pass(y_hbm, srcs, dsts, acc, src_v, dst_v, rows0, rows1,
              gsem0, gsem1, ssem0, ssem1, base_chunk, nblocks):
  def outer(g, carry):
    _agg_block(y_hbm, srcs, dsts, acc, src_v, dst_v, rows0, rows1,
               gsem0, gsem1, ssem0, ssem1, base_chunk + g * IB, IB)
    return carry

  lax.fori_loop(0, nblocks, outer, 0)


def _zero_acc(rows0, acc, s):
  def zrow(r, carry):
    for k in range(8):
      rows0[r, pl.ds(k * 16, 16)] = jnp.zeros((16,), jnp.float32)
    return carry

  lax.fori_loop(0, C, zrow, 0)
  for k in range(4):
    pltpu.sync_copy(rows0, acc.at[pl.ds(s * ZROWS + k * C, C)])
  pltpu.sync_copy(rows0.at[pl.ds(0, ZROWS - 4 * C)],
                  acc.at[pl.ds(s * ZROWS + 4 * C, ZROWS - 4 * C)])


def _writeout(acc, o, s):
  pltpu.sync_copy(acc.at[pl.ds(s * OROWS, OROWS)],
                  o.at[pl.ds(s * OROWS, OROWS)])


def _agg_edge_split_body(x_hbm, srcs, dsts, tsrc, tdst, q0, q1,
                         src_v, dst_v, rows0, rows1, acc,
                         gsem0, gsem1, ssem0, ssem1):
  c = lax.axis_index("c")
  s = lax.axis_index("s")
  w = c * 16 + s
  _zero_acc(rows0, acc, s)
  plsc.subcore_barrier()

  @pl.when(w < 31)
  def _():
    _agg_pass(x_hbm, srcs, dsts, acc, src_v, dst_v, rows0, rows1,
              gsem0, gsem1, ssem0, ssem1, w * CPW_ES, CPW_ES // IB)

  @pl.when(w == 31)
  def _():
    _agg_block(x_hbm, tsrc, tdst, acc, src_v, dst_v, rows0, rows1,
               gsem0, gsem1, ssem0, ssem1, 0, TAIL)

  plsc.subcore_barrier()

  @pl.when(c == 0)
  def _():
    _writeout(acc, q0, s)

  @pl.when(c == 1)
  def _():
    _writeout(acc, q1, s)


def _agg_col_split_body(ya, yb, srcs, dsts, tsrc, tdst, a0, a1,
                        src_v, dst_v, rows0, rows1, acc,
                        gsem0, gsem1, ssem0, ssem1):
  c = lax.axis_index("c")
  s = lax.axis_index("s")
  _zero_acc(rows0, acc, s)
  plsc.subcore_barrier()

  def cs_pass(y_hbm):
    @pl.when(s < 15)
    def _():
      _agg_pass(y_hbm, srcs, dsts, acc, src_v, dst_v, rows0, rows1,
                gsem0, gsem1, ssem0, ssem1, s * CPW_CS, CPW_CS // IB)

    @pl.when(s == 15)
    def _():
      _agg_pass(y_hbm, srcs, dsts, acc, src_v, dst_v, rows0, rows1,
                gsem0, gsem1, ssem0, ssem1, 15 * CPW_CS,
                (MAIN - 15 * CPW_CS) // IB)
      _agg_block(y_hbm, tsrc, tdst, acc, src_v, dst_v, rows0, rows1,
                 gsem0, gsem1, ssem0, ssem1, 0, TAIL)

  @pl.when(c == 0)
  def _():
    cs_pass(ya)

  @pl.when(c == 1)
  def _():
    cs_pass(yb)

  plsc.subcore_barrier()

  @pl.when(c == 0)
  def _():
    _writeout(acc, a0, s)

  @pl.when(c == 1)
  def _():
    _writeout(acc, a1, s)


def _make_agg(body):
  return pl.kernel(
      body,
      out_type=(jax.ShapeDtypeStruct((ACC_ROWS, 128), jnp.float32),
                jax.ShapeDtypeStruct((ACC_ROWS, 128), jnp.float32)),
      mesh=_MESH,
      scratch_types=(
          pltpu.VMEM((IB, C), jnp.int32),
          pltpu.VMEM((IB, C), jnp.int32),
          pltpu.VMEM((C, 128), jnp.float32),
          pltpu.VMEM((C, 128), jnp.float32),
          pltpu.VMEM_SHARED((ACC_ROWS, 128), jnp.float32),
          pltpu.SemaphoreType.DMA,
          pltpu.SemaphoreType.DMA,
          pltpu.SemaphoreType.DMA,
          pltpu.SemaphoreType.DMA,
      ),
  )


_agg_edge_split = _make_agg(_agg_edge_split_body)
_agg_col_split = _make_agg(_agg_col_split_body)

RB = 2000


def _mm1_body(p0, p1, w1, b1, ya, yb):
  h = p0[...] + p1[...]
  y = jnp.dot(h, w1[...], preferred_element_type=jnp.float32) + b1[...]
  y = jnp.maximum(y, 0.0)
  ya[...] = y[:, :128]
  yb[...] = y[:, 128:]


_mm1 = pl.pallas_call(
    _mm1_body,
    grid=(N // RB,),
    in_specs=[
        pl.BlockSpec((RB, D_IN), lambda i: (i, 0)),
        pl.BlockSpec((RB, D_IN), lambda i: (i, 0)),
        pl.BlockSpec((D_IN, D_HID), lambda i: (0, 0)),
        pl.BlockSpec((1, D_HID), lambda i: (0, 0)),
    ],
    out_specs=[
        pl.BlockSpec((RB, 128), lambda i: (i, 0)),
        pl.BlockSpec((RB, 128), lambda i: (i, 0)),
    ],
    out_shape=[
        jax.ShapeDtypeStruct((N, 128), jnp.float32),
        jax.ShapeDtypeStruct((N, 128), jnp.float32),
    ],
)


def _mm2_body(a0, a1, w2a, w2b, b2, w3, t3):
  h = (jnp.dot(a0[...], w2a[...], preferred_element_type=jnp.float32)
       + jnp.dot(a1[...], w2b[...], preferred_element_type=jnp.float32)
       + b2[...])
  h = jnp.maximum(h, 0.0)
  t3[...] = jnp.dot(h, w3[...], preferred_element_type=jnp.float32)


_mm2 = pl.pallas_call(
    _mm2_body,
    grid=(N // RB,),
    in_specs=[
        pl.BlockSpec((RB, 128), lambda i: (i, 0)),
        pl.BlockSpec((RB, 128), lambda i: (i, 0)),
        pl.BlockSpec((128, D_HID), lambda i: (0, 0)),
        pl.BlockSpec((128, D_HID), lambda i: (0, 0)),
        pl.BlockSpec((1, D_HID), lambda i: (0, 0)),
        pl.BlockSpec((D_HID, D_OUT), lambda i: (0, 0)),
    ],
    out_specs=pl.BlockSpec((RB, D_OUT), lambda i: (i, 0)),
    out_shape=jax.ShapeDtypeStruct((N, D_OUT), jnp.float32),
)


def _mm3_body(q0, q1, b3, out):
  out[...] = q0[...] + q1[...] + b3[...]


_mm3 = pl.pallas_call(
    _mm3_body,
    grid=(N // RB,),
    in_specs=[
        pl.BlockSpec((RB, D_OUT), lambda i: (i, 0)),
        pl.BlockSpec((RB, D_OUT), lambda i: (i, 0)),
        pl.BlockSpec((1, D_OUT), lambda i: (0, 0)),
    ],
    out_specs=pl.BlockSpec((RB, D_OUT), lambda i: (i, 0)),
    out_shape=jax.ShapeDtypeStruct((N, D_OUT), jnp.float32),
)


@jax.jit
def kernel(x, adj, W1, b1, W2, b2, W3, b3):
  srcs = adj[0].astype(jnp.int32).reshape(NCHUNK, C)
  dsts = adj[1].astype(jnp.int32).reshape(NCHUNK, C)
  npad = TAIL * C - (NCHUNK - MAIN) * C
  tsrc = jnp.concatenate([
      srcs[MAIN:].reshape(-1),
      jnp.arange(npad, dtype=jnp.int32) % N]).reshape(TAIL, C)
  tdst = jnp.concatenate([
      dsts[MAIN:].reshape(-1),
      N + jnp.arange(npad, dtype=jnp.int32) % (ACC_ROWS - N)]).reshape(TAIL, C)

  q0, q1 = _agg_edge_split(x, srcs, dsts, tsrc, tdst)
  ya, yb = _mm1(q0, q1, W1, b1.reshape(1, D_HID))
  a0, a1 = _agg_col_split(ya, yb, srcs, dsts, tsrc, tdst)
  t3 = _mm2(a0, a1, W2[:128], W2[128:], b2.reshape(1, D_HID), W3)
  r0, r1 = _agg_edge_split(t3, srcs, dsts, tsrc, tdst)
  return _mm3(r0, r1, b3.reshape(1, D_OUT))
